# baseline (speedup 1.0000x reference)
import functools
import jax
import jax.numpy as jnp
from jax import lax
from jax.experimental import pallas as pl
from jax.experimental.pallas import tpu as pltpu
from jax.experimental.pallas import tpu_sc as plsc

T = 4
B, N, C = 4, 2048, 1024
ROWS = B * N
NW = 32
RPW = ROWS // NW
R = 16
U = 2
L = 16

_mesh = plsc.VectorSubcoreMesh(core_axis_name="c", subcore_axis_name="s")


@functools.partial(
    pl.kernel,
    mesh=_mesh,
    out_type=[jax.ShapeDtypeStruct((ROWS * C,), jnp.float32)] * 4,
    scratch_types=[pltpu.VMEM((R * C,), jnp.float32) for _ in range(4)]
    + [pltpu.VMEM((R * L,), jnp.float32)]
    + [pltpu.SemaphoreType.DMA] * 5,
)
def _sc_agg(i0, i1, i2, i3, mpk, o0, o1, o2, o3, b0, b1, b2, b3, mb,
            s0, s1, s2, s3, sm):
    wid = lax.axis_index("s") * 2 + lax.axis_index("c")
    base = wid * RPW

    def chunk(ci, carry):
        r0 = base + ci * R
        h0 = pltpu.async_copy(i0.at[pl.ds(r0 * C, R * C)], b0, s0)
        h1 = pltpu.async_copy(i1.at[pl.ds(r0 * C, R * C)], b1, s1)
        h2 = pltpu.async_copy(i2.at[pl.ds(r0 * C, R * C)], b2, s2)
        h3 = pltpu.async_copy(i3.at[pl.ds(r0 * C, R * C)], b3, s3)
        hm = pltpu.async_copy(mpk.at[pl.ds(r0 * L, R * L)], mb, sm)
        h0.wait()
        h1.wait()
        h2.wait()
        h3.wait()
        hm.wait()

        @plsc.parallel_loop(0, R)
        def row(r):
            mv = mb[pl.ds(r * L, L)]
            idx = [jnp.full((L,), t, dtype=jnp.int32) for t in range(5)]
            m0 = mv.at[idx[0]].get(mode="promise_in_bounds")
            m1 = mv.at[idx[1]].get(mode="promise_in_bounds")
            m2 = mv.at[idx[2]].get(mode="promise_in_bounds")
            m3 = mv.at[idx[3]].get(mode="promise_in_bounds")
            g = mv.at[idx[4]].get(mode="promise_in_bounds")
            rcp = 1.0 / jnp.maximum(m0 + m1 + m2 + m3, 1.0)
            w0 = m0 * rcp
            w1 = m1 * rcp
            w2 = m2 * rcp
            w3 = m3 * rcp
            u0 = g * m0
            u1 = g * m1
            u2 = g * m2
            u3 = g * m3

            @plsc.parallel_loop(0, C // L, unroll=U)
            def sl(j):
                d = pl.ds(r * C + j * L, L)
                v0 = b0[d]
                v1 = b1[d]
                v2 = b2[d]
                v3 = b3[d]
                a = v0 * w0 + v1 * w1 + v2 * w2 + v3 * w3
                b0[d] = v0 + u0 * (a - v0)
                b1[d] = v1 + u1 * (a - v1)
                b2[d] = v2 + u2 * (a - v2)
                b3[d] = v3 + u3 * (a - v3)


        w0 = pltpu.async_copy(b0, o0.at[pl.ds(r0 * C, R * C)], s0)
        w1 = pltpu.async_copy(b1, o1.at[pl.ds(r0 * C, R * C)], s1)
        w2 = pltpu.async_copy(b2, o2.at[pl.ds(r0 * C, R * C)], s2)
        w3 = pltpu.async_copy(b3, o3.at[pl.ds(r0 * C, R * C)], s3)
        w0.wait()
        w1.wait()
        w2.wait()
        w3.wait()
        return carry

    lax.fori_loop(0, RPW // R, chunk, 0)


def kernel(out_0, out_1, out_2, out_3, mask_0, mask_1, mask_2, mask_3, agg_needed_mask):
    flat = [x.reshape(ROWS * C) for x in (out_0, out_1, out_2, out_3)]
    mcols = [m.astype(jnp.float32).reshape(ROWS, 1)
             for m in (mask_0, mask_1, mask_2, mask_3, agg_needed_mask)]
    mpk = jnp.concatenate(
        mcols + [jnp.zeros((ROWS, L - 5), jnp.float32)], axis=-1
    ).reshape(ROWS * L)
    res = _sc_agg(*flat, mpk)
    return tuple(r.reshape(B, N, C) for r in res)


# speedup vs baseline: 1.4237x; 1.4237x over previous
import functools
import jax
import jax.numpy as jnp
from jax import lax
from jax.experimental import pallas as pl
from jax.experimental.pallas import tpu as pltpu
from jax.experimental.pallas import tpu_sc as plsc

T = 4
B, N, C = 4, 2048, 1024
ROWS = B * N
ROW_BLOCK = 512
NW = 32
RPW = ROWS // NW
R = 16
U = 2
L = 16

_mesh = plsc.VectorSubcoreMesh(core_axis_name="c", subcore_axis_name="s")


@functools.partial(
    pl.kernel,
    mesh=_mesh,
    out_type=jax.ShapeDtypeStruct((ROWS * C,), jnp.float32),
    scratch_types=[pltpu.VMEM((R * C,), jnp.float32) for _ in range(4)]
    + [pltpu.VMEM((R * L,), jnp.float32)]
    + [pltpu.SemaphoreType.DMA] * 5,
)
def _sc_task3(i0, i1, i2, i3, mpk, o3, b0, b1, b2, b3, mb,
              s0, s1, s2, s3, sm):
    wid = lax.axis_index("s") * 2 + lax.axis_index("c")
    base = wid * RPW

    def chunk(ci, carry):
        r0 = base + ci * R
        h0 = pltpu.async_copy(i0.at[pl.ds(r0 * C, R * C)], b0, s0)
        h1 = pltpu.async_copy(i1.at[pl.ds(r0 * C, R * C)], b1, s1)
        h2 = pltpu.async_copy(i2.at[pl.ds(r0 * C, R * C)], b2, s2)
        h3 = pltpu.async_copy(i3.at[pl.ds(r0 * C, R * C)], b3, s3)
        hm = pltpu.async_copy(mpk.at[pl.ds(r0 * L, R * L)], mb, sm)
        h0.wait()
        h1.wait()
        h2.wait()
        h3.wait()
        hm.wait()

        @plsc.parallel_loop(0, R)
        def row(r):
            mv = mb[pl.ds(r * L, L)]
            idx = [jnp.full((L,), t, dtype=jnp.int32) for t in range(5)]
            m0 = mv.at[idx[0]].get(mode="promise_in_bounds")
            m1 = mv.at[idx[1]].get(mode="promise_in_bounds")
            m2 = mv.at[idx[2]].get(mode="promise_in_bounds")
            m3 = mv.at[idx[3]].get(mode="promise_in_bounds")
            g = mv.at[idx[4]].get(mode="promise_in_bounds")
            rcp = 1.0 / jnp.maximum(m0 + m1 + m2 + m3, 1.0)
            w0 = m0 * rcp
            w1 = m1 * rcp
            w2 = m2 * rcp
            w3 = m3 * rcp
            u3 = g * m3

            @plsc.parallel_loop(0, C // L, unroll=U)
            def sl(j):
                d = pl.ds(r * C + j * L, L)
                v0 = b0[d]
                v1 = b1[d]
                v2 = b2[d]
                v3 = b3[d]
                a = v0 * w0 + v1 * w1 + v2 * w2 + v3 * w3
                b3[d] = v3 + u3 * (a - v3)

        w3h = pltpu.async_copy(b3, o3.at[pl.ds(r0 * C, R * C)], s3)
        w3h.wait()
        return carry

    lax.fori_loop(0, RPW // R, chunk, 0)


def _agg_block_tc(o0, o1, o2, o3, m0, m1, m2, m3, g, n0, n1, n2):
    mm0 = m0[...]
    mm1 = m1[...]
    mm2 = m2[...]
    mm3 = m3[...]
    gg = g[...]
    s = o0[...] * mm0 + o1[...] * mm1 + o2[...] * mm2 + o3[...] * mm3
    cnt = jnp.maximum(mm0 + mm1 + mm2 + mm3, 1.0)
    aggregated = s / cnt
    n0[...] = jnp.where((gg * mm0) > 0, aggregated, o0[...])
    n1[...] = jnp.where((gg * mm1) > 0, aggregated, o1[...])
    n2[...] = jnp.where((gg * mm2) > 0, aggregated, o2[...])


def kernel(out_0, out_1, out_2, out_3, mask_0, mask_1, mask_2, mask_3, agg_needed_mask):
    outs = [x.reshape(ROWS, C) for x in (out_0, out_1, out_2, out_3)]
    cols = [m.astype(jnp.float32).reshape(ROWS, 1)
            for m in (mask_0, mask_1, mask_2, mask_3, agg_needed_mask)]

    grid = (ROWS // ROW_BLOCK,)
    row_spec = pl.BlockSpec((ROW_BLOCK, C), lambda i: (i, 0))
    col_spec = pl.BlockSpec((ROW_BLOCK, 1), lambda i: (i, 0))

    tc_res = pl.pallas_call(
        _agg_block_tc,
        grid=grid,
        in_specs=[row_spec] * 4 + [col_spec] * 5,
        out_specs=[row_spec] * 3,
        out_shape=[jax.ShapeDtypeStruct((ROWS, C), jnp.float32)] * 3,
        compiler_params=pltpu.CompilerParams(
            dimension_semantics=("parallel",),
            vmem_limit_bytes=100 * 1024 * 1024,
        ),
    )(*outs, *cols)

    flat = [x.reshape(ROWS * C) for x in outs]
    mpk = jnp.concatenate(
        cols + [jnp.zeros((ROWS, L - 5), jnp.float32)], axis=-1
    ).reshape(ROWS * L)
    sc3 = _sc_task3(*flat, mpk)

    return (tc_res[0].reshape(B, N, C), tc_res[1].reshape(B, N, C),
            tc_res[2].reshape(B, N, C), sc3.reshape(B, N, C))


# speedup vs baseline: 4.0018x; 2.8108x over previous
import jax
import jax.numpy as jnp
from jax.experimental import pallas as pl
from jax.experimental.pallas import tpu as pltpu

T = 4
B, N, C = 4, 2048, 1024
ROWS = B * N
ROW_BLOCK = 1024
COL_BLOCK = 512


def _agg_block(o0, o1, o2, o3, m0, m1, m2, m3, g,
               n0, n1, n2, n3):
    mm0 = m0[...]
    mm1 = m1[...]
    mm2 = m2[...]
    mm3 = m3[...]
    gg = g[...]
    s = o0[...] * mm0 + o1[...] * mm1 + o2[...] * mm2 + o3[...] * mm3
    cnt = jnp.maximum(mm0 + mm1 + mm2 + mm3, 1.0)
    aggregated = s / cnt
    n0[...] = jnp.where((gg * mm0) > 0, aggregated, o0[...])
    n1[...] = jnp.where((gg * mm1) > 0, aggregated, o1[...])
    n2[...] = jnp.where((gg * mm2) > 0, aggregated, o2[...])
    n3[...] = jnp.where((gg * mm3) > 0, aggregated, o3[...])


def kernel(out_0, out_1, out_2, out_3, mask_0, mask_1, mask_2, mask_3, agg_needed_mask):
    outs = [x.reshape(ROWS, C) for x in (out_0, out_1, out_2, out_3)]
    cols = [m.astype(jnp.float32).reshape(ROWS, 1)
            for m in (mask_0, mask_1, mask_2, mask_3, agg_needed_mask)]

    grid = (ROWS // ROW_BLOCK, C // COL_BLOCK)
    row_spec = pl.BlockSpec((ROW_BLOCK, COL_BLOCK), lambda i, j: (i, j))
    col_spec = pl.BlockSpec((ROW_BLOCK, 1), lambda i, j: (i, 0))

    res = pl.pallas_call(
        _agg_block,
        grid=grid,
        in_specs=[row_spec] * 4 + [col_spec] * 5,
        out_specs=[row_spec] * 4,
        out_shape=[jax.ShapeDtypeStruct((ROWS, C), jnp.float32)] * 4,
        compiler_params=pltpu.CompilerParams(
            dimension_semantics=("parallel", "arbitrary"),
            vmem_limit_bytes=100 * 1024 * 1024,
        ),
    )(*outs, *cols)
    return tuple(r.reshape(B, N, C) for r in res)


# speedup vs baseline: 4.0936x; 1.0229x over previous
import jax
import jax.numpy as jnp
from jax.experimental import pallas as pl
from jax.experimental.pallas import tpu as pltpu

T = 4
B, N, C = 4, 2048, 1024
ROWS = B * N
ROW_BLOCK = 512


def _agg_block(o0, o1, o2, o3, m0, m1, m2, m3, g,
               n0, n1, n2, n3):
    mm0 = m0[...]
    mm1 = m1[...]
    mm2 = m2[...]
    mm3 = m3[...]
    gg = g[...]
    s = o0[...] * mm0 + o1[...] * mm1 + o2[...] * mm2 + o3[...] * mm3
    cnt = jnp.maximum(mm0 + mm1 + mm2 + mm3, 1.0)
    aggregated = s / cnt
    n0[...] = jnp.where((gg * mm0) > 0, aggregated, o0[...])
    n1[...] = jnp.where((gg * mm1) > 0, aggregated, o1[...])
    n2[...] = jnp.where((gg * mm2) > 0, aggregated, o2[...])
    n3[...] = jnp.where((gg * mm3) > 0, aggregated, o3[...])


def kernel(out_0, out_1, out_2, out_3, mask_0, mask_1, mask_2, mask_3, agg_needed_mask):
    outs = [x.reshape(ROWS, C) for x in (out_0, out_1, out_2, out_3)]
    cols = [m.astype(jnp.float32).reshape(ROWS, 1)
            for m in (mask_0, mask_1, mask_2, mask_3, agg_needed_mask)]

    grid = (ROWS // ROW_BLOCK,)
    row_spec = pl.BlockSpec((ROW_BLOCK, C), lambda i: (i, 0))
    col_spec = pl.BlockSpec((ROW_BLOCK, 1), lambda i: (i, 0))

    res = pl.pallas_call(
        _agg_block,
        grid=grid,
        in_specs=[row_spec] * 4 + [col_spec] * 5,
        out_specs=[row_spec] * 4,
        out_shape=[jax.ShapeDtypeStruct((ROWS, C), jnp.float32)] * 4,
        compiler_params=pltpu.CompilerParams(
            dimension_semantics=("parallel",),
            vmem_limit_bytes=100 * 1024 * 1024,
        ),
    )(*outs, *cols)
    return tuple(r.reshape(B, N, C) for r in res)
